# baseline (speedup 1.0000x reference)
import functools

import jax
import jax.numpy as jnp
from jax import lax
from jax.experimental import pallas as pl
from jax.experimental.pallas import tpu as pltpu
from jax.experimental.pallas import tpu_sc as plsc

BATCH = 16384
NCLS = 1000
BETA = 1.0
EPS = 1e-10

NC = 2
NS = 16
L = 16
NW = NC * NS
ROWS_W = BATCH // NW
CHUNK = 64
NCH = ROWS_W // CHUNK
NFULL = NCLS // L
TAIL = NCLS - NFULL * L
LN2 = 0.6931471805599453

_mesh = plsc.VectorSubcoreMesh(
    core_axis_name="c", subcore_axis_name="s", num_cores=NC, num_subcores=NS
)


@functools.partial(
    pl.kernel,
    out_type=jax.ShapeDtypeStruct((NW, 4 * L), jnp.float32),
    mesh=_mesh,
    compiler_params=pltpu.CompilerParams(needs_layout_passes=False),
    scratch_types=[
        pltpu.VMEM((CHUNK, NCLS), jnp.float32),
        pltpu.VMEM((ROWS_W,), jnp.float32),
        pltpu.VMEM((ROWS_W,), jnp.float32),
        pltpu.VMEM((ROWS_W,), jnp.int32),
        pltpu.VMEM((ROWS_W,), jnp.int32),
        pltpu.VMEM((L,), jnp.float32),
        pltpu.VMEM((4 * L,), jnp.float32),
    ],
)
def _sc_stats(logits_hbm, labels_hbm, th_hbm, out_hbm,
              buf, zv, sv, pv, lv, thv, ov):
    wid = lax.axis_index("s") * NC + lax.axis_index("c")
    base = wid * ROWS_W
    pltpu.sync_copy(labels_hbm.at[pl.ds(base, ROWS_W)], lv)
    pltpu.sync_copy(th_hbm, thv)

    lanes = lax.iota(jnp.int32, L)
    tail_mask = lanes >= (L - TAIL)
    neg_inf = jnp.float32(-jnp.inf)
    zeros = jnp.zeros((L,), jnp.float32)

    def chunk_body(ch, _):
        pltpu.sync_copy(logits_hbm.at[pl.ds(base + ch * CHUNK, CHUNK)], buf)

        def group_body(g, _):
            def row_body(i, carry):
                gz, gs, gp = carry
                r = g * L + i

                def p1(j, vm):
                    return jnp.maximum(vm, buf[r, pl.ds(j * L, L)])
                vm = lax.fori_loop(0, NFULL, p1, jnp.full((L,), neg_inf))
                vt = buf[r, pl.ds(NCLS - L, L)]
                vm = jnp.maximum(vm, jnp.where(tail_mask, vt, neg_inf))
                m = jnp.max(vm)
                mv = jnp.full((L,), m)

                def p2(j, carry2):
                    az, asum, ai = carry2
                    v = buf[r, pl.ds(j * L, L)]
                    d = v - mv
                    e = jnp.exp(d)
                    cand = jnp.where(v == mv, j * L + lanes, jnp.int32(NCLS))
                    return az + e, asum + d * e, jnp.minimum(ai, cand)
                big = jnp.full((L,), NCLS, jnp.int32)
                az, asum, ai = lax.fori_loop(0, NFULL, p2,
                                             (zeros, zeros, big))
                d = vt - mv
                e = jnp.where(tail_mask, jnp.exp(d), 0.0)
                az = az + e
                asum = asum + d * e
                cand = jnp.where(tail_mask & (vt == mv),
                                 (NCLS - L) + lanes, jnp.int32(NCLS))
                ai = jnp.minimum(ai, cand)

                gz = jnp.where(lanes == i, jnp.full((L,), jnp.sum(az)), gz)
                gs = jnp.where(lanes == i, jnp.full((L,), jnp.sum(asum)), gs)
                gp = jnp.where(lanes == i,
                               jnp.full((L,), jnp.min(ai), jnp.int32), gp)
                return gz, gs, gp

            izero = jnp.zeros((L,), jnp.int32)
            gz, gs, gp = lax.fori_loop(0, L, row_body, (zeros, zeros, izero))
            row0 = ch * CHUNK + g * L
            zv[pl.ds(row0, L)] = gz
            sv[pl.ds(row0, L)] = gs
            pv[pl.ds(row0, L)] = gp
            return 0

        return lax.fori_loop(0, CHUNK // L, group_body, 0)

    lax.fori_loop(0, NCH, chunk_body, 0)

    th = thv[...]

    def fin(k, accs):
        a_ac, a_au, a_ic, a_iu = accs
        z = zv[pl.ds(k * L, L)]
        s = sv[pl.ds(k * L, L)]
        p = pv[pl.ds(k * L, L)]
        lab = lv[pl.ds(k * L, L)]
        conf = 1.0 / z
        zi = plsc.bitcast(z, jnp.int32)
        y = zi.astype(jnp.float32) * jnp.float32(LN2 / (1 << 23)) \
            - jnp.float32(127.0 * LN2)
        for _ in range(3):
            y = y - 1.0 + z * jnp.exp(-y)
        unc = y - s * conf
        t = 1.0 - 2.0 / (jnp.exp(2.0 * unc) + 1.0)
        acc = p == lab
        cert = unc <= th
        one_m_t = 1.0 - t
        one_m_c = 1.0 - conf
        a_ac = a_ac + jnp.where(acc & cert, conf * one_m_t, 0.0)
        a_au = a_au + jnp.where(acc & (~cert), conf * t, 0.0)
        a_ic = a_ic + jnp.where((~acc) & cert, one_m_c * one_m_t, 0.0)
        a_iu = a_iu + jnp.where((~acc) & (~cert), one_m_c * t, 0.0)
        return a_ac, a_au, a_ic, a_iu

    a_ac, a_au, a_ic, a_iu = lax.fori_loop(
        0, ROWS_W // L, fin, (zeros, zeros, zeros, zeros))
    ov[pl.ds(0, L)] = a_ac
    ov[pl.ds(L, L)] = a_au
    ov[pl.ds(2 * L, L)] = a_ic
    ov[pl.ds(3 * L, L)] = a_iu
    pltpu.sync_copy(ov, out_hbm.at[wid])


def _tc_finish(parts_ref, o_ref):
    x = parts_ref[...]
    n_ac = jnp.sum(x[:, 0:L])
    n_au = jnp.sum(x[:, L:2 * L])
    n_ic = jnp.sum(x[:, 2 * L:3 * L])
    n_iu = jnp.sum(x[:, 3 * L:4 * L])
    avu = (n_ac + n_iu) / (n_ac + n_au + n_ic + n_iu + EPS)
    o_ref[...] = jnp.full((1, 1), -BETA * jnp.log(avu + EPS))


def kernel(logits, labels, unc_th):
    labels32 = labels.astype(jnp.int32)
    th = jnp.full((L,), unc_th, jnp.float32)
    parts = _sc_stats(logits, labels32, th)
    loss = pl.pallas_call(
        _tc_finish,
        out_shape=jax.ShapeDtypeStruct((1, 1), jnp.float32),
    )(parts)
    return loss[0, 0]


# speedup vs baseline: 2.2859x; 2.2859x over previous
import functools

import jax
import jax.numpy as jnp
from jax import lax
from jax.experimental import pallas as pl
from jax.experimental.pallas import tpu as pltpu
from jax.experimental.pallas import tpu_sc as plsc

BATCH = 16384
NCLS = 1000
BETA = 1.0
EPS = 1e-10

NC = 2
NS = 16
L = 16
NW = NC * NS
ROWS_W = BATCH // NW
CHUNK = 64
NCH = ROWS_W // CHUNK
NFULL = NCLS // L
TAIL = NCLS - NFULL * L
LN2 = 0.6931471805599453

_mesh = plsc.VectorSubcoreMesh(
    core_axis_name="c", subcore_axis_name="s", num_cores=NC, num_subcores=NS
)


@functools.partial(
    pl.kernel,
    out_type=jax.ShapeDtypeStruct((NW, 4 * L), jnp.float32),
    mesh=_mesh,
    compiler_params=pltpu.CompilerParams(needs_layout_passes=False),
    scratch_types=[
        pltpu.VMEM((CHUNK, NCLS), jnp.float32),
        pltpu.VMEM((ROWS_W,), jnp.float32),
        pltpu.VMEM((ROWS_W,), jnp.float32),
        pltpu.VMEM((ROWS_W,), jnp.float32),
        pltpu.VMEM((ROWS_W,), jnp.int32),
        pltpu.VMEM((ROWS_W,), jnp.int32),
        pltpu.VMEM((L,), jnp.float32),
        pltpu.VMEM((4 * L,), jnp.float32),
    ],
)
def _sc_stats(logits_hbm, labels_hbm, th_hbm, out_hbm,
              buf, zv, sv, mvv, pv, lv, thv, ov):
    wid = lax.axis_index("s") * NC + lax.axis_index("c")
    base = wid * ROWS_W
    pltpu.sync_copy(labels_hbm.at[pl.ds(base, ROWS_W)], lv)
    pltpu.sync_copy(th_hbm, thv)

    lanes = lax.iota(jnp.int32, L)
    tail_mask = lanes >= (L - TAIL)
    neg_inf = jnp.float32(-jnp.inf)
    zeros = jnp.zeros((L,), jnp.float32)

    def chunk_body(ch, _):
        pltpu.sync_copy(logits_hbm.at[pl.ds(base + ch * CHUNK, CHUNK)], buf)

        def group_body(g, _):
            def row_body(i, carry):
                gz, gs, gm, gp = carry
                r = g * L + i

                def p(j, c2):
                    az, asum, vm, ai = c2
                    v = buf[r, pl.ds(j * L, L)]
                    e = jnp.exp(v)
                    win = v > vm
                    return (az + e, asum + v * e, jnp.maximum(vm, v),
                            jnp.where(win, j * L, ai))
                izero = jnp.zeros((L,), jnp.int32)
                az, asum, vm, ai = lax.fori_loop(
                    0, NFULL, p,
                    (zeros, zeros, jnp.full((L,), neg_inf), izero),
                    unroll=4)
                vt = buf[r, pl.ds(NCLS - L, L)]
                e = jnp.where(tail_mask, jnp.exp(vt), 0.0)
                az = az + e
                asum = asum + vt * e
                vtm = jnp.where(tail_mask, vt, neg_inf)
                win = vtm > vm
                ai = jnp.where(win, NCLS - L, ai)
                vm = jnp.maximum(vm, vtm)

                m = jnp.max(vm)
                cand = jnp.where(vm == jnp.full((L,), m), ai + lanes,
                                 jnp.int32(NCLS))
                gz = jnp.where(lanes == i, jnp.full((L,), jnp.sum(az)), gz)
                gs = jnp.where(lanes == i, jnp.full((L,), jnp.sum(asum)), gs)
                gm = jnp.where(lanes == i, jnp.full((L,), m), gm)
                gp = jnp.where(lanes == i,
                               jnp.full((L,), jnp.min(cand), jnp.int32), gp)
                return gz, gs, gm, gp

            izero = jnp.zeros((L,), jnp.int32)
            gz, gs, gm, gp = lax.fori_loop(
                0, L, row_body, (zeros, zeros, zeros, izero))
            row0 = ch * CHUNK + g * L
            zv[pl.ds(row0, L)] = gz
            sv[pl.ds(row0, L)] = gs
            mvv[pl.ds(row0, L)] = gm
            pv[pl.ds(row0, L)] = gp
            return 0

        return lax.fori_loop(0, CHUNK // L, group_body, 0)

    lax.fori_loop(0, NCH, chunk_body, 0)

    th = thv[...]

    def fin(k, accs):
        a_ac, a_au, a_ic, a_iu = accs
        z = zv[pl.ds(k * L, L)]
        s = sv[pl.ds(k * L, L)]
        mrow = mvv[pl.ds(k * L, L)]
        p = pv[pl.ds(k * L, L)]
        lab = lv[pl.ds(k * L, L)]
        conf = jnp.exp(mrow) / z
        zi = plsc.bitcast(z, jnp.int32)
        y = zi.astype(jnp.float32) * jnp.float32(LN2 / (1 << 23)) \
            - jnp.float32(127.0 * LN2)
        for _ in range(3):
            y = y - 1.0 + z * jnp.exp(-y)
        unc = y - s / z
        t = 1.0 - 2.0 / (jnp.exp(2.0 * unc) + 1.0)
        acc = p == lab
        cert = unc <= th
        one_m_t = 1.0 - t
        one_m_c = 1.0 - conf
        a_ac = a_ac + jnp.where(acc & cert, conf * one_m_t, 0.0)
        a_au = a_au + jnp.where(acc & (~cert), conf * t, 0.0)
        a_ic = a_ic + jnp.where((~acc) & cert, one_m_c * one_m_t, 0.0)
        a_iu = a_iu + jnp.where((~acc) & (~cert), one_m_c * t, 0.0)
        return a_ac, a_au, a_ic, a_iu

    a_ac, a_au, a_ic, a_iu = lax.fori_loop(
        0, ROWS_W // L, fin, (zeros, zeros, zeros, zeros))
    ov[pl.ds(0, L)] = a_ac
    ov[pl.ds(L, L)] = a_au
    ov[pl.ds(2 * L, L)] = a_ic
    ov[pl.ds(3 * L, L)] = a_iu
    pltpu.sync_copy(ov, out_hbm.at[wid])


def _tc_finish(parts_ref, o_ref):
    x = parts_ref[...]
    n_ac = jnp.sum(x[:, 0:L])
    n_au = jnp.sum(x[:, L:2 * L])
    n_ic = jnp.sum(x[:, 2 * L:3 * L])
    n_iu = jnp.sum(x[:, 3 * L:4 * L])
    avu = (n_ac + n_iu) / (n_ac + n_au + n_ic + n_iu + EPS)
    o_ref[...] = jnp.full((1, 1), -BETA * jnp.log(avu + EPS))


def kernel(logits, labels, unc_th):
    labels32 = labels.astype(jnp.int32)
    th = jnp.full((L,), unc_th, jnp.float32)
    parts = _sc_stats(logits, labels32, th)
    loss = pl.pallas_call(
        _tc_finish,
        out_shape=jax.ShapeDtypeStruct((1, 1), jnp.float32),
    )(parts)
    return loss[0, 0]
